# speedup vs baseline: 1.6016x; 1.6016x over previous
import functools

import jax
import jax.numpy as jnp
from jax import lax
from jax.experimental import pallas as pl
from jax.experimental.pallas import tpu as pltpu
from jax.experimental.pallas import tpu_sc as plsc

HIDDEN = 1024
EPS = 1e-12

_NC = 2
_NS = 16
_NW = _NC * _NS


def _make_sc_gather(n_tok: int, ch: int):
    n_per_w = n_tok // _NW
    nch = n_per_w // ch
    mesh = plsc.VectorSubcoreMesh(core_axis_name="c", subcore_axis_name="s")

    @functools.partial(
        pl.kernel,
        mesh=mesh,
        out_type=jax.ShapeDtypeStruct((n_tok, HIDDEN), jnp.float32),
        scratch_types=[
            pltpu.VMEM((n_per_w,), jnp.int32),
            pltpu.VMEM((ch, HIDDEN), jnp.float32),
            pltpu.SemaphoreType.DMA,
        ],
    )
    def gather_k(ids_hbm, table_hbm, out_hbm, ids_v, rows_v, sem):
        wid = lax.axis_index("s") * _NC + lax.axis_index("c")
        base = wid * n_per_w
        pltpu.sync_copy(ids_hbm.at[pl.ds(base, n_per_w)], ids_v)

        def body(c, carry):
            off = pl.multiple_of(c * ch, 8)
            pltpu.async_copy(
                table_hbm.at[ids_v.at[pl.ds(off, ch)]], rows_v, sem
            ).wait()
            pltpu.sync_copy(rows_v, out_hbm.at[pl.ds(base + off, ch)])
            return carry

        lax.fori_loop(0, nch, body, 0)

    return gather_k


def _make_tc_ln(n_tok: int, seq_len: int, blk: int):
    grid = n_tok // blk
    pos_blocks = seq_len // blk

    def body(x_ref, pos_ref, type_ref, g_ref, b_ref, o_ref):
        x = x_ref[...] + pos_ref[...] + type_ref[0:1, :]
        mu = jnp.mean(x, axis=-1, keepdims=True)
        xc = x - mu
        var = jnp.mean(xc * xc, axis=-1, keepdims=True)
        rstd = lax.rsqrt(var + EPS)
        o_ref[...] = (xc * rstd) * g_ref[0:1, :] + b_ref[0:1, :]

    return pl.pallas_call(
        body,
        grid=(grid,),
        in_specs=[
            pl.BlockSpec((blk, HIDDEN), lambda i: (i, 0)),
            pl.BlockSpec((blk, HIDDEN), lambda i: (i % pos_blocks, 0)),
            pl.BlockSpec((2, HIDDEN), lambda i: (0, 0)),
            pl.BlockSpec((1, HIDDEN), lambda i: (0, 0)),
            pl.BlockSpec((1, HIDDEN), lambda i: (0, 0)),
        ],
        out_specs=pl.BlockSpec((blk, HIDDEN), lambda i: (i, 0)),
        out_shape=jax.ShapeDtypeStruct((n_tok, HIDDEN), jnp.float32),
    )


def kernel(input_ids, word_table, pos_table, type_table, ln_gamma, ln_beta):
    B, L = input_ids.shape
    n_tok = B * L
    ids = input_ids.reshape(n_tok).astype(jnp.int32)
    rows = _make_sc_gather(n_tok, ch=64)(ids, word_table)
    out = _make_tc_ln(n_tok, L, blk=256)(
        rows,
        pos_table,
        type_table,
        ln_gamma.reshape(1, HIDDEN),
        ln_beta.reshape(1, HIDDEN),
    )
    return out.reshape(B, L, HIDDEN)


# speedup vs baseline: 1.8215x; 1.1373x over previous
import functools

import jax
import jax.numpy as jnp
from jax import lax
from jax.experimental import pallas as pl
from jax.experimental.pallas import tpu as pltpu
from jax.experimental.pallas import tpu_sc as plsc

HIDDEN = 1024
EPS = 1e-12

_NC = 2
_NS = 16
_NW = _NC * _NS


def _make_sc_gather(n_tok: int, ch: int):
    n_per_w = n_tok // _NW
    nch = n_per_w // ch
    mesh = plsc.VectorSubcoreMesh(core_axis_name="c", subcore_axis_name="s")

    @functools.partial(
        pl.kernel,
        mesh=mesh,
        out_type=jax.ShapeDtypeStruct((n_tok, HIDDEN), jnp.float32),
        scratch_types=[
            pltpu.VMEM((n_per_w,), jnp.int32),
            pltpu.VMEM((ch, HIDDEN), jnp.float32),
            pltpu.SemaphoreType.DMA,
        ],
    )
    def gather_k(ids_hbm, table_hbm, out_hbm, ids_v, rows_v, sem):
        wid = lax.axis_index("s") * _NC + lax.axis_index("c")
        base = wid * n_per_w
        pltpu.sync_copy(ids_hbm.at[pl.ds(base, n_per_w)], ids_v)

        def body(c, carry):
            off = pl.multiple_of(c * ch, 8)
            pltpu.async_copy(
                table_hbm.at[ids_v.at[pl.ds(off, ch)]], rows_v, sem
            ).wait()
            pltpu.sync_copy(rows_v, out_hbm.at[pl.ds(base + off, ch)])
            return carry

        lax.fori_loop(0, nch, body, 0)

    return gather_k


def _make_tc_ln(n_tok: int, seq_len: int, batch: int, blk: int):
    pos_blocks = seq_len // blk

    def body(x_ref, pos_ref, type_ref, g_ref, b_ref, o_ref):
        x = x_ref[...] + pos_ref[...] + type_ref[0:1, :]
        mu = jnp.mean(x, axis=-1, keepdims=True)
        xc = x - mu
        var = jnp.mean(xc * xc, axis=-1, keepdims=True)
        rstd = lax.rsqrt(var + EPS)
        o_ref[...] = (xc * rstd) * g_ref[0:1, :] + b_ref[0:1, :]

    return pl.pallas_call(
        body,
        grid=(pos_blocks, batch),
        in_specs=[
            pl.BlockSpec((blk, HIDDEN), lambda p, b: (b * pos_blocks + p, 0)),
            pl.BlockSpec((blk, HIDDEN), lambda p, b: (p, 0)),
            pl.BlockSpec((2, HIDDEN), lambda p, b: (0, 0)),
            pl.BlockSpec((1, HIDDEN), lambda p, b: (0, 0)),
            pl.BlockSpec((1, HIDDEN), lambda p, b: (0, 0)),
        ],
        out_specs=pl.BlockSpec((blk, HIDDEN), lambda p, b: (b * pos_blocks + p, 0)),
        out_shape=jax.ShapeDtypeStruct((n_tok, HIDDEN), jnp.float32),
    )


def kernel(input_ids, word_table, pos_table, type_table, ln_gamma, ln_beta):
    B, L = input_ids.shape
    n_tok = B * L
    ids = input_ids.reshape(n_tok).astype(jnp.int32)
    rows = _make_sc_gather(n_tok, ch=64)(ids, word_table)
    out = _make_tc_ln(n_tok, L, B, blk=512)(
        rows,
        pos_table,
        type_table,
        ln_gamma.reshape(1, HIDDEN),
        ln_beta.reshape(1, HIDDEN),
    )
    return out.reshape(B, L, HIDDEN)
